# Initial kernel scaffold; baseline (speedup 1.0000x reference)
import jax
import jax.numpy as jnp
from jax.experimental import pallas as pl


def kernel(x, edge_index, W1, b1, W2, b2, W3, b3):
    raise NotImplementedError("write your pallas kernel here")


# speedup vs baseline: 13.6540x; 13.6540x over previous
import functools

import jax
import jax.numpy as jnp
from jax import lax
from jax.experimental import pallas as pl
from jax.experimental.pallas import tpu as pltpu
from jax.experimental.pallas import tpu_sc as plsc

_N = 10000
_E = 320000
_NPAD = 10240
_RPT = _NPAD // 16
_CH = 128
_NCHUNK = 80
_TILES = 32
_EPAD = _TILES * _NCHUNK * _CH
_DC = 16

_mesh = plsc.VectorSubcoreMesh(
    core_axis_name="c", subcore_axis_name="s", num_cores=2, num_subcores=16)


def _make_agg(d):
  @functools.partial(
      pl.kernel,
      out_type=jax.ShapeDtypeStruct((2, _NPAD, d), jnp.float32),
      mesh=_mesh,
      compiler_params=pltpu.CompilerParams(use_tc_tiling_on_sc=False),
      scratch_types=[
          pltpu.VMEM((_NCHUNK, _CH), jnp.int32),
          pltpu.VMEM((_NCHUNK, _CH), jnp.int32),
          pltpu.VMEM((_CH, d), jnp.float32),
          pltpu.VMEM((_RPT, d), jnp.float32),
          pltpu.VMEM_SHARED((_NPAD, d), jnp.float32),
          pltpu.SemaphoreType.DMA,
      ],
  )
  def agg(g_hbm, src_hbm, dst_hbm, zeros_hbm, out_hbm,
          src_v, dst_v, rows_v, stage_v, acc_sh, sem):
    c = lax.axis_index("c")
    s = lax.axis_index("s")
    t = c * 16 + s
    row0 = s * _RPT
    pltpu.sync_copy(zeros_hbm, stage_v)
    pltpu.sync_copy(stage_v, acc_sh.at[pl.ds(row0, _RPT)])
    pltpu.sync_copy(src_hbm.at[t], src_v)
    pltpu.sync_copy(dst_hbm.at[t], dst_v)
    plsc.subcore_barrier()

    def body(j, carry):
      pltpu.async_copy(g_hbm.at[src_v.at[j]], rows_v, sem).wait()
      pltpu.sync_copy(rows_v, acc_sh.at[dst_v.at[j]], add=True)
      return carry

    lax.fori_loop(0, _NCHUNK, body, 0)
    plsc.subcore_barrier()
    pltpu.sync_copy(acc_sh.at[pl.ds(row0, _RPT)], stage_v)
    pltpu.sync_copy(stage_v, out_hbm.at[c].at[pl.ds(row0, _RPT)])

  return agg


@functools.partial(
    pl.kernel,
    out_type=jax.ShapeDtypeStruct((2, _NPAD, _DC), jnp.float32),
    mesh=_mesh,
    compiler_params=pltpu.CompilerParams(use_tc_tiling_on_sc=False),
    scratch_types=[
        pltpu.VMEM((_NCHUNK, _CH), jnp.int32),
        pltpu.VMEM((_CH, _DC), jnp.float32),
        pltpu.VMEM((_RPT, _DC), jnp.float32),
        pltpu.VMEM_SHARED((_NPAD, _DC), jnp.float32),
    ],
)
def _deg_kernel(dst_hbm, ones_hbm, zeros_hbm, out_hbm,
                dst_v, ones_v, stage_v, acc_sh):
  c = lax.axis_index("c")
  s = lax.axis_index("s")
  t = c * 16 + s
  row0 = s * _RPT
  pltpu.sync_copy(zeros_hbm, stage_v)
  pltpu.sync_copy(stage_v, acc_sh.at[pl.ds(row0, _RPT)])
  pltpu.sync_copy(dst_hbm.at[t], dst_v)
  pltpu.sync_copy(ones_hbm, ones_v)
  plsc.subcore_barrier()

  def body(j, carry):
    pltpu.sync_copy(ones_v, acc_sh.at[dst_v.at[j]], add=True)
    return carry

  lax.fori_loop(0, _NCHUNK, body, 0)
  plsc.subcore_barrier()
  pltpu.sync_copy(acc_sh.at[pl.ds(row0, _RPT)], stage_v)
  pltpu.sync_copy(stage_v, out_hbm.at[c].at[pl.ds(row0, _RPT)])


_agg64 = _make_agg(64)
_agg16 = _make_agg(16)


def _prep_body(cnt_ref, x_ref, w1_ref, dis_ref, hw_ref, g_ref):
  cnt16 = cnt_ref[0, : _N, :] + cnt_ref[1, : _N, :]
  dis = jnp.max(lax.rsqrt(cnt16 + 1.0), axis=1, keepdims=True)
  hw = jnp.dot(x_ref[...], w1_ref[...], preferred_element_type=jnp.float32)
  dis_ref[...] = dis
  hw_ref[...] = hw
  g_ref[...] = hw * dis


def _layer_body(parts_ref, hw_ref, dis_ref, b_ref, wn_ref, hwn_ref, gn_ref):
  dis = dis_ref[...]
  agg = parts_ref[0, : _N, :] + parts_ref[1, : _N, :]
  h = dis * agg + (dis * dis) * hw_ref[...] + b_ref[...]
  h = jnp.maximum(h, 0.0)
  hwn = jnp.dot(h, wn_ref[...], preferred_element_type=jnp.float32)
  hwn_ref[...] = hwn
  gn_ref[...] = hwn * dis


def _final_body(parts_ref, hw_ref, dis_ref, b_ref, out_ref):
  dis = dis_ref[...]
  agg = parts_ref[0, : _N, :] + parts_ref[1, : _N, :]
  o = dis * agg + (dis * dis) * hw_ref[...] + b_ref[...]
  col = lax.broadcasted_iota(jnp.int32, (_N, 16), 1)
  mask = col < 6
  om = jnp.where(mask, o, jnp.float32(-1e30))
  m = jnp.max(om, axis=1, keepdims=True)
  ex = jnp.where(mask, jnp.exp(o - m), 0.0)
  lse = m + jnp.log(jnp.sum(ex, axis=1, keepdims=True))
  out_ref[...] = o - lse


_f32 = jnp.float32

_prep_call = pl.pallas_call(
    _prep_body,
    out_shape=(
        jax.ShapeDtypeStruct((_N, 1), _f32),
        jax.ShapeDtypeStruct((_N, 64), _f32),
        jax.ShapeDtypeStruct((_N, 64), _f32),
    ),
)

_layer64_call = pl.pallas_call(
    _layer_body,
    out_shape=(
        jax.ShapeDtypeStruct((_N, 64), _f32),
        jax.ShapeDtypeStruct((_N, 64), _f32),
    ),
)

_layer16_call = pl.pallas_call(
    _layer_body,
    out_shape=(
        jax.ShapeDtypeStruct((_N, 16), _f32),
        jax.ShapeDtypeStruct((_N, 16), _f32),
    ),
)

_final_call = pl.pallas_call(
    _final_body,
    out_shape=jax.ShapeDtypeStruct((_N, 16), _f32),
)


def kernel(x, edge_index, W1, b1, W2, b2, W3, b3):
  src = edge_index[0]
  dst = edge_index[1]
  pad = _EPAD - _E
  srcp = jnp.concatenate([src, jnp.zeros((pad,), jnp.int32)]).reshape(
      _TILES, _NCHUNK, _CH)
  dstp = jnp.concatenate([dst, jnp.full((pad,), _N, jnp.int32)]).reshape(
      _TILES, _NCHUNK, _CH)
  zeros64 = jnp.zeros((_RPT, 64), _f32)
  zeros16 = jnp.zeros((_RPT, 16), _f32)
  ones16 = jnp.ones((_CH, _DC), _f32)
  W3p = jnp.pad(W3, ((0, 0), (0, 10)))
  b3p = jnp.pad(b3, (0, 10)).reshape(1, 16)

  cnt = _deg_kernel(dstp, ones16, zeros16)
  dis, hw1, g1 = _prep_call(cnt, x, W1)
  p1 = _agg64(g1, srcp, dstp, zeros64)
  hw2, g2 = _layer64_call(p1, hw1, dis, b1.reshape(1, 64), W2)
  p2 = _agg64(g2, srcp, dstp, zeros64)
  hw3, g3 = _layer16_call(p2, hw2, dis, b2.reshape(1, 64), W3p)
  p3 = _agg16(g3, srcp, dstp, zeros16)
  out16 = _final_call(p3, hw3, dis, b3p)
  return out16[:, :6]


# speedup vs baseline: 15.6800x; 1.1484x over previous
import functools

import jax
import jax.numpy as jnp
from jax import lax
from jax.experimental import pallas as pl
from jax.experimental.pallas import tpu as pltpu
from jax.experimental.pallas import tpu_sc as plsc

_N = 10000
_E = 320000
_NPAD = 10240
_RPT = _NPAD // 16
_CH = 128
_NCHUNK = 80
_TILES = 32
_EPAD = _TILES * _NCHUNK * _CH
_DC = 8

_mesh = plsc.VectorSubcoreMesh(
    core_axis_name="c", subcore_axis_name="s", num_cores=2, num_subcores=16)


_NB = 4


def _make_agg(d):
  @functools.partial(
      pl.kernel,
      out_type=jax.ShapeDtypeStruct((2, _NPAD, d), jnp.float32),
      mesh=_mesh,
      compiler_params=pltpu.CompilerParams(use_tc_tiling_on_sc=False),
      scratch_types=[
          pltpu.VMEM((_NCHUNK, _CH), jnp.int32),
          pltpu.VMEM((_NCHUNK, _CH), jnp.int32),
          pltpu.VMEM((_NB, _CH, d), jnp.float32),
          pltpu.VMEM_SHARED((_NPAD, d), jnp.float32),
          pltpu.SemaphoreType.DMA,
      ],
  )
  def agg(g_hbm, src_hbm, dst_hbm, zeros_hbm, out_hbm,
          src_v, dst_v, rows_v, acc_sh, gsem):
    c = lax.axis_index("c")
    s = lax.axis_index("s")
    t = c * 16 + s
    row0 = s * _RPT
    pltpu.sync_copy(zeros_hbm, acc_sh.at[pl.ds(row0, _RPT)])
    pltpu.sync_copy(src_hbm.at[t], src_v)
    pltpu.sync_copy(dst_hbm.at[t], dst_v)
    plsc.subcore_barrier()

    for b in range(_NB):
      pltpu.async_copy(g_hbm.at[src_v.at[b]], rows_v.at[b], gsem)

    def outer(o, carry):
      for b in range(_NB):
        j = o * _NB + b
        pltpu.make_async_copy(g_hbm.at[src_v.at[j]], rows_v.at[b], gsem).wait()
        pltpu.sync_copy(rows_v.at[b], acc_sh.at[dst_v.at[j]], add=True)

        @pl.when(j + _NB < _NCHUNK)
        def _():
          pltpu.async_copy(g_hbm.at[src_v.at[j + _NB]], rows_v.at[b], gsem)

      return carry

    lax.fori_loop(0, _NCHUNK // _NB, outer, 0)
    plsc.subcore_barrier()
    pltpu.sync_copy(acc_sh.at[pl.ds(row0, _RPT)], out_hbm.at[c].at[pl.ds(row0, _RPT)])

  return agg


@functools.partial(
    pl.kernel,
    out_type=jax.ShapeDtypeStruct((2, _NPAD, _DC), jnp.float32),
    mesh=_mesh,
    compiler_params=pltpu.CompilerParams(use_tc_tiling_on_sc=False),
    scratch_types=[
        pltpu.VMEM((_NCHUNK, _CH), jnp.int32),
        pltpu.VMEM((_CH, _DC), jnp.float32),
        pltpu.VMEM_SHARED((_NPAD, _DC), jnp.float32),
    ],
)
def _deg_kernel(dst_hbm, ones_hbm, zeros_hbm, out_hbm,
                dst_v, ones_v, acc_sh):
  c = lax.axis_index("c")
  s = lax.axis_index("s")
  t = c * 16 + s
  row0 = s * _RPT
  pltpu.sync_copy(zeros_hbm, acc_sh.at[pl.ds(row0, _RPT)])
  pltpu.sync_copy(dst_hbm.at[t], dst_v)
  pltpu.sync_copy(ones_hbm, ones_v)
  plsc.subcore_barrier()

  def body(j, carry):
    pltpu.sync_copy(ones_v, acc_sh.at[dst_v.at[j]], add=True)
    return carry

  lax.fori_loop(0, _NCHUNK, body, 0)
  plsc.subcore_barrier()
  pltpu.sync_copy(acc_sh.at[pl.ds(row0, _RPT)], out_hbm.at[c].at[pl.ds(row0, _RPT)])


_agg64 = _make_agg(64)
_agg16 = _make_agg(16)


def _prep_body(cnt_ref, x_ref, w1_ref, dis_ref, hw_ref, g_ref):
  cnt16 = cnt_ref[0, : _N, :] + cnt_ref[1, : _N, :]
  dis = jnp.max(lax.rsqrt(cnt16 + 1.0), axis=1, keepdims=True)
  hw = jnp.dot(x_ref[...], w1_ref[...], preferred_element_type=jnp.float32)
  dis_ref[...] = dis
  hw_ref[...] = hw
  g_ref[...] = hw * dis


def _layer_body(parts_ref, hw_ref, dis_ref, b_ref, wn_ref, hwn_ref, gn_ref):
  dis = dis_ref[...]
  agg = parts_ref[0, : _N, :] + parts_ref[1, : _N, :]
  h = dis * agg + (dis * dis) * hw_ref[...] + b_ref[...]
  h = jnp.maximum(h, 0.0)
  hwn = jnp.dot(h, wn_ref[...], preferred_element_type=jnp.float32)
  hwn_ref[...] = hwn
  gn_ref[...] = hwn * dis


def _final_body(parts_ref, hw_ref, dis_ref, b_ref, out_ref):
  dis = dis_ref[...]
  agg = parts_ref[0, : _N, :] + parts_ref[1, : _N, :]
  o = dis * agg + (dis * dis) * hw_ref[...] + b_ref[...]
  col = lax.broadcasted_iota(jnp.int32, (_N, 16), 1)
  mask = col < 6
  om = jnp.where(mask, o, jnp.float32(-1e30))
  m = jnp.max(om, axis=1, keepdims=True)
  ex = jnp.where(mask, jnp.exp(o - m), 0.0)
  lse = m + jnp.log(jnp.sum(ex, axis=1, keepdims=True))
  out_ref[...] = o - lse


_f32 = jnp.float32

_prep_call = pl.pallas_call(
    _prep_body,
    out_shape=(
        jax.ShapeDtypeStruct((_N, 1), _f32),
        jax.ShapeDtypeStruct((_N, 64), _f32),
        jax.ShapeDtypeStruct((_N, 64), _f32),
    ),
)

_layer64_call = pl.pallas_call(
    _layer_body,
    out_shape=(
        jax.ShapeDtypeStruct((_N, 64), _f32),
        jax.ShapeDtypeStruct((_N, 64), _f32),
    ),
)

_layer16_call = pl.pallas_call(
    _layer_body,
    out_shape=(
        jax.ShapeDtypeStruct((_N, 16), _f32),
        jax.ShapeDtypeStruct((_N, 16), _f32),
    ),
)

_final_call = pl.pallas_call(
    _final_body,
    out_shape=jax.ShapeDtypeStruct((_N, 16), _f32),
)


def kernel(x, edge_index, W1, b1, W2, b2, W3, b3):
  src = edge_index[0]
  dst = edge_index[1]
  pad = _EPAD - _E
  srcp = jnp.concatenate([src, jnp.zeros((pad,), jnp.int32)]).reshape(
      _TILES, _NCHUNK, _CH)
  dstp = jnp.concatenate([dst, jnp.full((pad,), _N, jnp.int32)]).reshape(
      _TILES, _NCHUNK, _CH)
  zeros64 = jnp.zeros((_RPT, 64), _f32)
  zeros16 = jnp.zeros((_RPT, 16), _f32)
  onesd = jnp.ones((_CH, _DC), _f32)
  zerosd = jnp.zeros((_RPT, _DC), _f32)
  W3p = jnp.pad(W3, ((0, 0), (0, 10)))
  b3p = jnp.pad(b3, (0, 10)).reshape(1, 16)

  cnt = _deg_kernel(dstp, onesd, zerosd)
  dis, hw1, g1 = _prep_call(cnt, x, W1)
  p1 = _agg64(g1, srcp, dstp, zeros64)
  hw2, g2 = _layer64_call(p1, hw1, dis, b1.reshape(1, 64), W2)
  p2 = _agg64(g2, srcp, dstp, zeros64)
  hw3, g3 = _layer16_call(p2, hw2, dis, b2.reshape(1, 64), W3p)
  p3 = _agg16(g3, srcp, dstp, zeros16)
  out16 = _final_call(p3, hw3, dis, b3p)
  return out16[:, :6]


# speedup vs baseline: 15.7328x; 1.0034x over previous
import functools

import jax
import jax.numpy as jnp
from jax import lax
from jax.experimental import pallas as pl
from jax.experimental.pallas import tpu as pltpu
from jax.experimental.pallas import tpu_sc as plsc

_N = 10000
_E = 320000
_NPAD = 10240
_RPT = _NPAD // 16
_CH = 128
_NCHUNK = 80
_TILES = 32
_EPAD = _TILES * _NCHUNK * _CH
_DC = 8

_mesh = plsc.VectorSubcoreMesh(
    core_axis_name="c", subcore_axis_name="s", num_cores=2, num_subcores=16)


_NB = 4


def _make_agg(d):
  @functools.partial(
      pl.kernel,
      out_type=jax.ShapeDtypeStruct((2, _NPAD, d), jnp.float32),
      mesh=_mesh,
      compiler_params=pltpu.CompilerParams(use_tc_tiling_on_sc=False),
      scratch_types=[
          pltpu.VMEM((_NCHUNK, _CH), jnp.int32),
          pltpu.VMEM((_NCHUNK, _CH), jnp.int32),
          pltpu.VMEM((_NB, _CH, d), jnp.float32),
          pltpu.VMEM_SHARED((_NPAD, d), jnp.float32),
          pltpu.SemaphoreType.DMA,
      ],
  )
  def agg(g_hbm, src_hbm, dst_hbm, zeros_hbm, out_hbm,
          src_v, dst_v, rows_v, acc_sh, gsem):
    c = lax.axis_index("c")
    s = lax.axis_index("s")
    t = c * 16 + s
    row0 = s * _RPT
    pltpu.sync_copy(zeros_hbm, acc_sh.at[pl.ds(row0, _RPT)])
    pltpu.sync_copy(src_hbm.at[t], src_v)
    pltpu.sync_copy(dst_hbm.at[t], dst_v)
    plsc.subcore_barrier()

    for b in range(_NB):
      pltpu.async_copy(g_hbm.at[src_v.at[b]], rows_v.at[b], gsem)

    def outer(o, carry):
      for b in range(_NB):
        j = o * _NB + b
        pltpu.make_async_copy(g_hbm.at[src_v.at[j]], rows_v.at[b], gsem).wait()
        pltpu.sync_copy(rows_v.at[b], acc_sh.at[dst_v.at[j]], add=True)

        @pl.when(j + _NB < _NCHUNK)
        def _():
          pltpu.async_copy(g_hbm.at[src_v.at[j + _NB]], rows_v.at[b], gsem)

      return carry

    lax.fori_loop(0, _NCHUNK // _NB, outer, 0)
    plsc.subcore_barrier()
    pltpu.sync_copy(acc_sh.at[pl.ds(row0, _RPT)], out_hbm.at[c].at[pl.ds(row0, _RPT)])

  return agg


@functools.partial(
    pl.kernel,
    out_type=jax.ShapeDtypeStruct((2, _NPAD, _DC), jnp.float32),
    mesh=_mesh,
    compiler_params=pltpu.CompilerParams(use_tc_tiling_on_sc=False),
    scratch_types=[
        pltpu.VMEM((_NCHUNK, _CH), jnp.int32),
        pltpu.VMEM((_CH, _DC), jnp.float32),
        pltpu.VMEM_SHARED((_NPAD, _DC), jnp.float32),
    ],
)
def _deg_kernel(dst_hbm, ones_hbm, zeros_hbm, out_hbm,
                dst_v, ones_v, acc_sh):
  c = lax.axis_index("c")
  s = lax.axis_index("s")
  t = c * 16 + s
  row0 = s * _RPT
  pltpu.sync_copy(zeros_hbm, acc_sh.at[pl.ds(row0, _RPT)])
  pltpu.sync_copy(dst_hbm.at[t], dst_v)
  pltpu.sync_copy(ones_hbm, ones_v)
  plsc.subcore_barrier()

  def body(j, carry):
    pltpu.sync_copy(ones_v, acc_sh.at[dst_v.at[j]], add=True)
    return carry

  lax.fori_loop(0, _NCHUNK, body, 0)
  plsc.subcore_barrier()
  pltpu.sync_copy(acc_sh.at[pl.ds(row0, _RPT)], out_hbm.at[c].at[pl.ds(row0, _RPT)])


_agg64 = _make_agg(64)
_agg16 = _make_agg(16)


def _prep_body(cnt_ref, x_ref, w1_ref, dis_ref, hw_ref, g_ref):
  cnt16 = cnt_ref[0, : _N, :] + cnt_ref[1, : _N, :]
  dis = jnp.max(lax.rsqrt(cnt16 + 1.0), axis=1, keepdims=True)
  hw = jnp.dot(x_ref[...], w1_ref[...], preferred_element_type=jnp.float32)
  dis_ref[...] = dis
  hw_ref[...] = hw
  g_ref[...] = hw * dis


def _layer_body(parts_ref, hw_ref, dis_ref, b_ref, wn_ref, hwn_ref, gn_ref):
  dis = dis_ref[...]
  agg = parts_ref[0, : _N, :] + parts_ref[1, : _N, :]
  h = dis * agg + (dis * dis) * hw_ref[...] + b_ref[...]
  h = jnp.maximum(h, 0.0)
  hwn = jnp.dot(h, wn_ref[...], preferred_element_type=jnp.float32)
  hwn_ref[...] = hwn
  gn_ref[...] = hwn * dis


def _final_body(parts_ref, hw_ref, dis_ref, b_ref, out_ref):
  dis = dis_ref[...]
  agg = parts_ref[0, : _N, :] + parts_ref[1, : _N, :]
  o = dis * agg + (dis * dis) * hw_ref[...] + b_ref[...]
  col = lax.broadcasted_iota(jnp.int32, (_N, 16), 1)
  mask = col < 6
  om = jnp.where(mask, o, jnp.float32(-1e30))
  m = jnp.max(om, axis=1, keepdims=True)
  ex = jnp.where(mask, jnp.exp(o - m), 0.0)
  lse = m + jnp.log(jnp.sum(ex, axis=1, keepdims=True))
  out_ref[...] = o - lse


_f32 = jnp.float32

_prep_call = pl.pallas_call(
    _prep_body,
    out_shape=(
        jax.ShapeDtypeStruct((_N, 1), _f32),
        jax.ShapeDtypeStruct((_N, 64), _f32),
        jax.ShapeDtypeStruct((_N, 64), _f32),
    ),
)

_layer64_call = pl.pallas_call(
    _layer_body,
    out_shape=(
        jax.ShapeDtypeStruct((_N, 64), _f32),
        jax.ShapeDtypeStruct((_N, 64), _f32),
    ),
)

_layer16_call = pl.pallas_call(
    _layer_body,
    out_shape=(
        jax.ShapeDtypeStruct((_N, 16), _f32),
        jax.ShapeDtypeStruct((_N, 16), _f32),
    ),
)

_final_call = pl.pallas_call(
    _final_body,
    out_shape=jax.ShapeDtypeStruct((_N, 16), _f32),
)


def kernel(x, edge_index, W1, b1, W2, b2, W3, b3):
  src = edge_index[0]
  dst = edge_index[1]
  pad = _EPAD - _E
  srcp = jnp.concatenate([src, jnp.zeros((pad,), jnp.int32)]).reshape(
      _TILES, _NCHUNK, _CH)
  dpad = _N + jnp.arange(pad, dtype=jnp.int32) % (_NPAD - _N)
  dstp = jnp.concatenate([dst, dpad]).reshape(_TILES, _NCHUNK, _CH)
  zeros64 = jnp.zeros((_RPT, 64), _f32)
  zeros16 = jnp.zeros((_RPT, 16), _f32)
  onesd = jnp.ones((_CH, _DC), _f32)
  zerosd = jnp.zeros((_RPT, _DC), _f32)
  W3p = jnp.pad(W3, ((0, 0), (0, 10)))
  b3p = jnp.pad(b3, (0, 10)).reshape(1, 16)

  cnt = _deg_kernel(dstp, onesd, zerosd)
  dis, hw1, g1 = _prep_call(cnt, x, W1)
  p1 = _agg64(g1, srcp, dstp, zeros64)
  hw2, g2 = _layer64_call(p1, hw1, dis, b1.reshape(1, 64), W2)
  p2 = _agg64(g2, srcp, dstp, zeros64)
  hw3, g3 = _layer16_call(p2, hw2, dis, b2.reshape(1, 64), W3p)
  p3 = _agg16(g3, srcp, dstp, zeros16)
  out16 = _final_call(p3, hw3, dis, b3p)
  return out16[:, :6]


# speedup vs baseline: 34.8116x; 2.2127x over previous
import functools

import jax
import jax.numpy as jnp
from jax import lax
from jax.experimental import pallas as pl
from jax.experimental.pallas import tpu as pltpu
from jax.experimental.pallas import tpu_sc as plsc

_N = 10000
_E = 320000
_NPAD = 10112
_RPT = _NPAD // 16
_CH = 128
_NCHUNK = 80
_TILES = 32
_EPAD = _TILES * _NCHUNK * _CH
_DC = 8
_NB = 2

_mesh = plsc.VectorSubcoreMesh(
    core_axis_name="c", subcore_axis_name="s", num_cores=2, num_subcores=16)


def _make_agg(d):
  @functools.partial(
      pl.kernel,
      out_type=jax.ShapeDtypeStruct((2, _NPAD, d), jnp.float32),
      mesh=_mesh,
      compiler_params=pltpu.CompilerParams(use_tc_tiling_on_sc=False),
      scratch_types=[
          pltpu.VMEM((_NCHUNK, _CH), jnp.int32),
          pltpu.VMEM((_NCHUNK, _CH), jnp.int32),
          pltpu.VMEM((_NB, _CH, d), jnp.float32),
          pltpu.VMEM_SHARED((_NPAD, d), jnp.float32),
          pltpu.VMEM_SHARED((_NPAD, d), jnp.float32),
          pltpu.SemaphoreType.DMA,
      ],
  )
  def agg(g_hbm, src_hbm, dst_hbm, zeros_hbm, out_hbm,
          src_v, dst_v, rows_v, g_sh, acc_sh, gsem):
    c = lax.axis_index("c")
    s = lax.axis_index("s")
    t = c * 16 + s
    row0 = s * _RPT
    pltpu.sync_copy(g_hbm.at[pl.ds(row0, _RPT)], g_sh.at[pl.ds(row0, _RPT)])
    pltpu.sync_copy(zeros_hbm, acc_sh.at[pl.ds(row0, _RPT)])
    pltpu.sync_copy(src_hbm.at[t], src_v)
    pltpu.sync_copy(dst_hbm.at[t], dst_v)
    plsc.subcore_barrier()

    for b in range(_NB):
      pltpu.async_copy(g_sh.at[src_v.at[b]], rows_v.at[b], gsem)

    def outer(o, carry):
      for b in range(_NB):
        j = o * _NB + b
        pltpu.make_async_copy(g_sh.at[src_v.at[j]], rows_v.at[b], gsem).wait()
        pltpu.sync_copy(rows_v.at[b], acc_sh.at[dst_v.at[j]], add=True)

        @pl.when(j + _NB < _NCHUNK)
        def _():
          pltpu.async_copy(g_sh.at[src_v.at[j + _NB]], rows_v.at[b], gsem)

      return carry

    lax.fori_loop(0, _NCHUNK // _NB, outer, 0)
    plsc.subcore_barrier()
    pltpu.sync_copy(acc_sh.at[pl.ds(row0, _RPT)],
                    out_hbm.at[c].at[pl.ds(row0, _RPT)])

  return agg


@functools.partial(
    pl.kernel,
    out_type=jax.ShapeDtypeStruct((2, _NPAD, _DC), jnp.float32),
    mesh=_mesh,
    compiler_params=pltpu.CompilerParams(use_tc_tiling_on_sc=False),
    scratch_types=[
        pltpu.VMEM((_NCHUNK, _CH), jnp.int32),
        pltpu.VMEM((_CH, _DC), jnp.float32),
        pltpu.VMEM_SHARED((_NPAD, _DC), jnp.float32),
    ],
)
def _deg_kernel(dst_hbm, ones_hbm, zeros_hbm, out_hbm, dst_v, ones_v, acc_sh):
  c = lax.axis_index("c")
  s = lax.axis_index("s")
  t = c * 16 + s
  row0 = s * _RPT
  pltpu.sync_copy(zeros_hbm, acc_sh.at[pl.ds(row0, _RPT)])
  pltpu.sync_copy(dst_hbm.at[t], dst_v)
  pltpu.sync_copy(ones_hbm, ones_v)
  plsc.subcore_barrier()

  def body(j, carry):
    pltpu.sync_copy(ones_v, acc_sh.at[dst_v.at[j]], add=True)
    return carry

  lax.fori_loop(0, _NCHUNK, body, 0)
  plsc.subcore_barrier()
  pltpu.sync_copy(acc_sh.at[pl.ds(row0, _RPT)],
                  out_hbm.at[c].at[pl.ds(row0, _RPT)])


_agg64 = _make_agg(64)
_agg16 = _make_agg(16)


def _prep_body(cnt_ref, x_ref, w1_ref, dis_ref, hw_ref, g_ref):
  cnt = cnt_ref[0, : _N, :] + cnt_ref[1, : _N, :]
  dis = jnp.max(lax.rsqrt(cnt + 1.0), axis=1, keepdims=True)
  hw = jnp.dot(x_ref[...], w1_ref[...], preferred_element_type=jnp.float32)
  dis_ref[...] = dis
  hw_ref[...] = hw
  g_ref[: _N, :] = hw * dis
  g_ref[_N :, :] = jnp.zeros((_NPAD - _N, 64), jnp.float32)


def _layer_body(parts_ref, hw_ref, dis_ref, b_ref, wn_ref, hwn_ref, gn_ref, *,
                dn):
  dis = dis_ref[...]
  agg = parts_ref[0, : _N, :] + parts_ref[1, : _N, :]
  h = dis * agg + (dis * dis) * hw_ref[...] + b_ref[...]
  h = jnp.maximum(h, 0.0)
  hwn = jnp.dot(h, wn_ref[...], preferred_element_type=jnp.float32)
  hwn_ref[...] = hwn
  gn_ref[: _N, :] = hwn * dis
  gn_ref[_N :, :] = jnp.zeros((_NPAD - _N, dn), jnp.float32)


def _final_body(parts_ref, hw_ref, dis_ref, b_ref, out_ref):
  dis = dis_ref[...]
  agg = parts_ref[0, : _N, :] + parts_ref[1, : _N, :]
  o = dis * agg + (dis * dis) * hw_ref[...] + b_ref[...]
  col = lax.broadcasted_iota(jnp.int32, (_N, 16), 1)
  mask = col < 6
  om = jnp.where(mask, o, jnp.float32(-1e30))
  m = jnp.max(om, axis=1, keepdims=True)
  ex = jnp.where(mask, jnp.exp(o - m), 0.0)
  lse = m + jnp.log(jnp.sum(ex, axis=1, keepdims=True))
  out_ref[...] = o - lse


_f32 = jnp.float32

_prep_call = pl.pallas_call(
    _prep_body,
    out_shape=(
        jax.ShapeDtypeStruct((_N, 1), _f32),
        jax.ShapeDtypeStruct((_N, 64), _f32),
        jax.ShapeDtypeStruct((_NPAD, 64), _f32),
    ),
)

_layer64_call = pl.pallas_call(
    functools.partial(_layer_body, dn=64),
    out_shape=(
        jax.ShapeDtypeStruct((_N, 64), _f32),
        jax.ShapeDtypeStruct((_NPAD, 64), _f32),
    ),
)

_layer16_call = pl.pallas_call(
    functools.partial(_layer_body, dn=16),
    out_shape=(
        jax.ShapeDtypeStruct((_N, 16), _f32),
        jax.ShapeDtypeStruct((_NPAD, 16), _f32),
    ),
)

_final_call = pl.pallas_call(
    _final_body,
    out_shape=jax.ShapeDtypeStruct((_N, 16), _f32),
)


def kernel(x, edge_index, W1, b1, W2, b2, W3, b3):
  src = edge_index[0]
  dst = edge_index[1]
  pad = _EPAD - _E
  srcp = jnp.concatenate([src, jnp.zeros((pad,), jnp.int32)]).reshape(
      _TILES, _NCHUNK, _CH)
  dpad = _N + jnp.arange(pad, dtype=jnp.int32) % (_NPAD - _N)
  dstp = jnp.concatenate([dst, dpad]).reshape(_TILES, _NCHUNK, _CH)
  zeros64 = jnp.zeros((_RPT, 64), _f32)
  zeros16 = jnp.zeros((_RPT, 16), _f32)
  onesd = jnp.ones((_CH, _DC), _f32)
  zerosd = jnp.zeros((_RPT, _DC), _f32)
  W3p = jnp.pad(W3, ((0, 0), (0, 10)))
  b3p = jnp.pad(b3, (0, 10)).reshape(1, 16)

  cnt = _deg_kernel(dstp, onesd, zerosd)
  dis, hw1, g1 = _prep_call(cnt, x, W1)
  p1 = _agg64(g1, srcp, dstp, zeros64)
  hw2, g2 = _layer64_call(p1, hw1, dis, b1.reshape(1, 64), W2)
  p2 = _agg64(g2, srcp, dstp, zeros64)
  hw3, g3 = _layer16_call(p2, hw2, dis, b2.reshape(1, 64), W3p)
  p3 = _agg16(g3, srcp, dstp, zeros16)
  out16 = _final_call(p3, hw3, dis, b3p)
  return out16[:, :6]


# speedup vs baseline: 37.3759x; 1.0737x over previous
import functools

import jax
import jax.numpy as jnp
from jax import lax
from jax.experimental import pallas as pl
from jax.experimental.pallas import tpu as pltpu
from jax.experimental.pallas import tpu_sc as plsc

_N = 10000
_E = 320000
_NPAD = 10112
_RPT = _NPAD // 16
_CH = 128
_NCHUNK = 80
_TILES = 32
_EPAD = _TILES * _NCHUNK * _CH
_DC = 8
_NR = 4

_mesh = plsc.VectorSubcoreMesh(
    core_axis_name="c", subcore_axis_name="s", num_cores=2, num_subcores=16)


def _make_agg(d):
  @functools.partial(
      pl.kernel,
      out_type=jax.ShapeDtypeStruct((2, _NPAD, d), jnp.float32),
      mesh=_mesh,
      compiler_params=pltpu.CompilerParams(use_tc_tiling_on_sc=False),
      scratch_types=[
          pltpu.VMEM((_NCHUNK, _CH), jnp.int32),
          pltpu.VMEM((_NR, _CH), jnp.int32),
          pltpu.VMEM((_NR, _CH, d), jnp.float32),
          pltpu.VMEM_SHARED((_NPAD, d), jnp.float32),
          pltpu.VMEM_SHARED((_NPAD, d), jnp.float32),
          pltpu.SemaphoreType.DMA,
          pltpu.SemaphoreType.DMA,
          pltpu.SemaphoreType.DMA,
      ],
  )
  def agg(g_hbm, src_hbm, dst_hbm, zeros_hbm, out_hbm,
          src_v, dring_v, rows_v, g_sh, acc_sh, gsem, ssem, isem):
    c = lax.axis_index("c")
    s = lax.axis_index("s")
    t = c * 16 + s
    row0 = s * _RPT
    pltpu.sync_copy(g_hbm.at[pl.ds(row0, _RPT)], g_sh.at[pl.ds(row0, _RPT)])
    pltpu.sync_copy(zeros_hbm, acc_sh.at[pl.ds(row0, _RPT)])
    pltpu.sync_copy(src_hbm.at[t], src_v)
    plsc.subcore_barrier()

    def issue(j, b):
      pltpu.async_copy(dst_hbm.at[t].at[j], dring_v.at[b], isem)
      pltpu.async_copy(g_sh.at[src_v.at[j]], rows_v.at[b], gsem)

    def scat(b):
      return pltpu.make_async_copy(
          rows_v.at[b], acc_sh.at[dring_v.at[b]], ssem)

    for b in range(2):
      issue(b, b)

    def outer(o, carry):
      for b4 in range(_NR):
        j = o * _NR + b4
        pltpu.make_async_copy(dst_hbm.at[t].at[j], dring_v.at[b4], isem).wait()
        pltpu.make_async_copy(g_sh.at[src_v.at[j]], rows_v.at[b4], gsem).wait()
        pltpu.async_copy(rows_v.at[b4], acc_sh.at[dring_v.at[b4]], ssem,
                         add=True)

        @pl.when(j >= 2)
        def _():
          scat((b4 - 2) % _NR).wait()

        @pl.when(j + 2 < _NCHUNK)
        def _():
          issue(j + 2, (b4 + 2) % _NR)

      return carry

    lax.fori_loop(0, _NCHUNK // _NR, outer, 0)
    scat((_NCHUNK - 2) % _NR).wait()
    scat((_NCHUNK - 1) % _NR).wait()
    plsc.subcore_barrier()
    pltpu.sync_copy(acc_sh.at[pl.ds(row0, _RPT)],
                    out_hbm.at[c].at[pl.ds(row0, _RPT)])

  return agg


@functools.partial(
    pl.kernel,
    out_type=jax.ShapeDtypeStruct((2, _NPAD, _DC), jnp.float32),
    mesh=_mesh,
    compiler_params=pltpu.CompilerParams(use_tc_tiling_on_sc=False),
    scratch_types=[
        pltpu.VMEM((_NCHUNK, _CH), jnp.int32),
        pltpu.VMEM((_CH, _DC), jnp.float32),
        pltpu.VMEM_SHARED((_NPAD, _DC), jnp.float32),
        pltpu.SemaphoreType.DMA,
    ],
)
def _deg_kernel(dst_hbm, ones_hbm, zeros_hbm, out_hbm,
                dst_v, ones_v, acc_sh, ssem):
  c = lax.axis_index("c")
  s = lax.axis_index("s")
  t = c * 16 + s
  row0 = s * _RPT
  pltpu.sync_copy(zeros_hbm, acc_sh.at[pl.ds(row0, _RPT)])
  pltpu.sync_copy(dst_hbm.at[t], dst_v)
  pltpu.sync_copy(ones_hbm, ones_v)
  plsc.subcore_barrier()

  def body(j, carry):
    pltpu.async_copy(ones_v, acc_sh.at[dst_v.at[j]], ssem, add=True)

    @pl.when(j >= 8)
    def _():
      pltpu.make_async_copy(ones_v, acc_sh.at[dst_v.at[j]], ssem).wait()

    return carry

  lax.fori_loop(0, _NCHUNK, body, 0)

  def drain(j, carry):
    pltpu.make_async_copy(ones_v, acc_sh.at[dst_v.at[j]], ssem).wait()
    return carry

  lax.fori_loop(0, 8, drain, 0)
  plsc.subcore_barrier()
  pltpu.sync_copy(acc_sh.at[pl.ds(row0, _RPT)],
                  out_hbm.at[c].at[pl.ds(row0, _RPT)])


_agg64 = _make_agg(64)
_agg16 = _make_agg(16)


def _mm1_body(x_ref, w1_ref, hw_ref):
  hw_ref[...] = jnp.dot(x_ref[...], w1_ref[...],
                        preferred_element_type=jnp.float32)


def _scale_body(cnt_ref, hw_ref, dis_ref, g_ref):
  cnt = cnt_ref[0, : _N, :] + cnt_ref[1, : _N, :]
  dis = jnp.max(lax.rsqrt(cnt + 1.0), axis=1, keepdims=True)
  dis_ref[...] = dis
  g_ref[: _N, :] = hw_ref[...] * dis
  g_ref[_N :, :] = jnp.zeros((_NPAD - _N, 64), jnp.float32)


def _layer_body(parts_ref, hw_ref, dis_ref, b_ref, wn_ref, hwn_ref, gn_ref, *,
                dn):
  dis = dis_ref[...]
  agg = parts_ref[0, : _N, :] + parts_ref[1, : _N, :]
  h = dis * agg + (dis * dis) * hw_ref[...] + b_ref[...]
  h = jnp.maximum(h, 0.0)
  hwn = jnp.dot(h, wn_ref[...], preferred_element_type=jnp.float32)
  hwn_ref[...] = hwn
  gn_ref[: _N, :] = hwn * dis
  gn_ref[_N :, :] = jnp.zeros((_NPAD - _N, dn), jnp.float32)


def _final_body(parts_ref, hw_ref, dis_ref, b_ref, out_ref):
  dis = dis_ref[...]
  agg = parts_ref[0, : _N, :] + parts_ref[1, : _N, :]
  o = dis * agg + (dis * dis) * hw_ref[...] + b_ref[...]
  col = lax.broadcasted_iota(jnp.int32, (_N, 16), 1)
  mask = col < 6
  om = jnp.where(mask, o, jnp.float32(-1e30))
  m = jnp.max(om, axis=1, keepdims=True)
  ex = jnp.where(mask, jnp.exp(o - m), 0.0)
  lse = m + jnp.log(jnp.sum(ex, axis=1, keepdims=True))
  out_ref[...] = o - lse


_f32 = jnp.float32

_mm1_call = pl.pallas_call(
    _mm1_body,
    out_shape=jax.ShapeDtypeStruct((_N, 64), _f32),
)

_scale_call = pl.pallas_call(
    _scale_body,
    out_shape=(
        jax.ShapeDtypeStruct((_N, 1), _f32),
        jax.ShapeDtypeStruct((_NPAD, 64), _f32),
    ),
)

_layer64_call = pl.pallas_call(
    functools.partial(_layer_body, dn=64),
    out_shape=(
        jax.ShapeDtypeStruct((_N, 64), _f32),
        jax.ShapeDtypeStruct((_NPAD, 64), _f32),
    ),
)

_layer16_call = pl.pallas_call(
    functools.partial(_layer_body, dn=16),
    out_shape=(
        jax.ShapeDtypeStruct((_N, 16), _f32),
        jax.ShapeDtypeStruct((_NPAD, 16), _f32),
    ),
)

_final_call = pl.pallas_call(
    _final_body,
    out_shape=jax.ShapeDtypeStruct((_N, 16), _f32),
)


def kernel(x, edge_index, W1, b1, W2, b2, W3, b3):
  src = edge_index[0]
  dst = edge_index[1]
  pad = _EPAD - _E
  srcp = jnp.concatenate([src, jnp.zeros((pad,), jnp.int32)]).reshape(
      _TILES, _NCHUNK, _CH)
  dpad = _N + jnp.arange(pad, dtype=jnp.int32) % (_NPAD - _N)
  dstp = jnp.concatenate([dst, dpad]).reshape(_TILES, _NCHUNK, _CH)
  zeros64 = jnp.zeros((_RPT, 64), _f32)
  zeros16 = jnp.zeros((_RPT, 16), _f32)
  onesd = jnp.ones((_CH, _DC), _f32)
  zerosd = jnp.zeros((_RPT, _DC), _f32)
  W3p = jnp.pad(W3, ((0, 0), (0, 10)))
  b3p = jnp.pad(b3, (0, 10)).reshape(1, 16)

  cnt = _deg_kernel(dstp, onesd, zerosd)
  hw1 = _mm1_call(x, W1)
  dis, g1 = _scale_call(cnt, hw1)
  p1 = _agg64(g1, srcp, dstp, zeros64)
  hw2, g2 = _layer64_call(p1, hw1, dis, b1.reshape(1, 64), W2)
  p2 = _agg64(g2, srcp, dstp, zeros64)
  hw3, g3 = _layer16_call(p2, hw2, dis, b2.reshape(1, 64), W3p)
  p3 = _agg16(g3, srcp, dstp, zeros16)
  out16 = _final_call(p3, hw3, dis, b3p)
  return out16[:, :6]
